# baseline (speedup 1.0000x reference)
import functools

import jax
import jax.numpy as jnp
from jax import lax
from jax.experimental import pallas as pl
from jax.experimental.pallas import tpu as pltpu
from jax.experimental.pallas import tpu_sc as plsc

N = 10000
E = 320000
D = 128
H = 64
C = 40
K = 10

NP = 10240
NLOC = NP // 2
TROWS = NLOC // 16
RCH = 40
NCH = TROWS // RCH
ECH = 128
CAPC = 104
CAPE = CAPC * ECH
RAWCH = 256
NRAW = 20480 // RAWCH
TRASH = NLOC
ZROW = NP - 1


def _sc_body(x_hbm, rawr_hbm, rawc_hbm, temp_hbm,
             hid_hbm, u_hbm, s_hbm,
             rawbr, rawbc, row_v, colstage, col_v, gbuf, gbuf2,
             abuf, sbuf, dis2b, tempv, offr, gsem, gsem2, ssem, ssem2, xsem,
             acc_sp):
    c = lax.axis_index("c")
    tid = lax.axis_index("s")
    base = tid * TROWS
    gbase = c * NLOC + base
    i32 = jnp.int32

    ones = jnp.full((16,), 1.0, jnp.float32)
    half = jnp.full((16,), 0.5, jnp.float32)

    def babylonian_sqrt(d):
        y = half * (ones + d)
        for _it in range(12):
            y = half * (y + d / y)
        return y

    def cross_sc_sync():
        plsc.subcore_barrier()
        @pl.when(tid == 0)
        def _():
            pltpu.semaphore_signal(xsem, 1, core_index=1 - c)
            pltpu.semaphore_wait(xsem, 1)
        plsc.subcore_barrier()

    pltpu.sync_copy(temp_hbm, tempv)

    zrow_v = jnp.full((16,), ZROW, i32)
    trash_v = jnp.full((16,), TRASH, i32)
    @pl.loop(0, CAPE // 16)
    def _(i):
        sl = pl.ds(i * 16, 16)
        row_v[sl] = zrow_v
        colstage[sl] = trash_v

    lo_v = jnp.full((16,), c * NLOC, i32)
    hi_v = jnp.full((16,), c * NLOC + NLOC, i32)

    offr[0] = jnp.asarray(0, i32)
    @pl.loop(0, NRAW)
    def _bucket_step(q):
        pltpu.sync_copy(rawr_hbm.at[tid, q], rawbr)
        pltpu.sync_copy(rawc_hbm.at[tid, q], rawbc)
        @pl.loop(0, RAWCH // 16)
        def _inner(i):
            sl = pl.ds(i * 16, 16)
            colv = rawbc[sl]
            roww = rawbr[sl]
            m = (colv >= lo_v) & (colv < hi_v)
            cnt = plsc.all_reduce_population_count(m)[0]
            off = offr[0]
            offc = jnp.minimum(off, CAPE - 16)
            plsc.store_compressed(row_v.at[pl.ds(offc, 16)], roww, mask=m)
            plsc.store_compressed(
                colstage.at[pl.ds(offc, 16)], colv - lo_v, mask=m)
            offr[0] = off + cnt

    off = offr[0]
    nch = lax.div(off + (ECH - 1), jnp.asarray(ECH, i32))
    npair = jnp.maximum(lax.div(nch + 1, jnp.asarray(2, i32)), 1)

    @pl.loop(0, CAPC)
    def _(j):
        for g in range(ECH // 16):
            col_v[j, pl.ds(g * 16, 16)] = colstage[pl.ds(j * ECH + g * 16, 16)]

    @pl.loop(0, RCH)
    def _(i):
        z = jnp.zeros((16,), jnp.float32)
        for g in range(D // 16):
            abuf[i, pl.ds(g * 16, 16)] = z
    @pl.loop(0, NCH)
    def _(jj):
        pltpu.sync_copy(abuf, acc_sp.at[pl.ds(base + jj * RCH, RCH)])
    plsc.subcore_barrier()
    @pl.loop(0, ECH)
    def _(i):
        for g in range(D // 16):
            gbuf[i, pl.ds(g * 16, 16)] = ones
    @pl.loop(0, nch)
    def _(j):
        pltpu.sync_copy(gbuf, acc_sp.at[col_v.at[j]], add=True)
    plsc.subcore_barrier()

    t0v = tempv[0, :]
    @pl.loop(0, NCH)
    def _(jj):
        r0l = base + jj * RCH
        r0g = gbase + jj * RCH
        pltpu.sync_copy(acc_sp.at[pl.ds(r0l, RCH)], abuf)
        pltpu.sync_copy(x_hbm.at[pl.ds(r0g, RCH)], sbuf)
        @pl.loop(0, RCH)
        def _(i):
            d = abuf[i, pl.ds(0, 16)] + ones
            d2 = ones / d
            dis2b[jj * RCH + i, :] = d2
            dv = ones / babylonian_sqrt(d)
            for g in range(D // 16):
                sl = pl.ds(g * 16, 16)
                un = dv * sbuf[i, sl]
                abuf[i, sl] = un
                sbuf[i, sl] = t0v * un
        pltpu.sync_copy(abuf, u_hbm.at[pl.ds(r0g, RCH)])
        pltpu.sync_copy(abuf, acc_sp.at[pl.ds(r0l, RCH)])
        pltpu.sync_copy(sbuf, s_hbm.at[pl.ds(r0g, RCH)])
    cross_sc_sync()

    def _gather_start(j, buf, sem):
        pltpu.async_copy(u_hbm.at[row_v.at[pl.ds(j * ECH, ECH)]], buf, sem)

    def _gather_wait(buf, sem):
        pltpu.make_async_copy(
            u_hbm.at[row_v.at[pl.ds(0, ECH)]], buf, sem).wait()

    def _scatter_start(j, buf, sem):
        pltpu.async_copy(buf, acc_sp.at[col_v.at[j]], sem, add=True)

    def _scatter_wait(buf, sem):
        pltpu.make_async_copy(buf, acc_sp.at[col_v.at[0]], sem).wait()

    for k in range(K):
        _gather_start(0, gbuf, gsem)
        _gather_start(1, gbuf2, gsem2)
        @pl.loop(0, npair - 1)
        def _(j2):
            b = 2 * j2
            _gather_wait(gbuf, gsem)
            _scatter_start(b, gbuf, ssem)
            _gather_wait(gbuf2, gsem2)
            _scatter_start(b + 1, gbuf2, ssem2)
            _scatter_wait(gbuf, ssem)
            _gather_start(b + 2, gbuf, gsem)
            _scatter_wait(gbuf2, ssem2)
            _gather_start(b + 3, gbuf2, gsem2)
        bl = 2 * (npair - 1)
        _gather_wait(gbuf, gsem)
        _scatter_start(bl, gbuf, ssem)
        _gather_wait(gbuf2, gsem2)
        _scatter_start(bl + 1, gbuf2, ssem2)
        _scatter_wait(gbuf, ssem)
        _scatter_wait(gbuf2, ssem2)
        plsc.subcore_barrier()

        last = k == K - 1
        tkv = tempv[k + 1, :]
        @pl.loop(0, NCH)
        def _(jj):
            r0l = base + jj * RCH
            r0g = gbase + jj * RCH
            pltpu.sync_copy(acc_sp.at[pl.ds(r0l, RCH)], abuf)
            pltpu.sync_copy(s_hbm.at[pl.ds(r0g, RCH)], sbuf)
            @pl.loop(0, RCH)
            def _(i):
                d2 = dis2b[jj * RCH + i, :]
                if last:
                    iv = babylonian_sqrt(ones / d2)
                for g in range(D // 16):
                    sl = pl.ds(g * 16, 16)
                    un = d2 * abuf[i, sl]
                    s = sbuf[i, sl] + tkv * un
                    if last:
                        s = iv * s
                    else:
                        abuf[i, sl] = un
                    sbuf[i, sl] = s
            if last:
                pltpu.sync_copy(sbuf, hid_hbm.at[pl.ds(r0g, RCH)])
            else:
                pltpu.sync_copy(abuf, u_hbm.at[pl.ds(r0g, RCH)])
                pltpu.sync_copy(abuf, acc_sp.at[pl.ds(r0l, RCH)])
                pltpu.sync_copy(sbuf, s_hbm.at[pl.ds(r0g, RCH)])
        if not last:
            cross_sc_sync()


def _propagate(x_flat, rawr, rawc, temp_b):
    mesh = plsc.VectorSubcoreMesh(core_axis_name="c", subcore_axis_name="s")
    f32 = jnp.float32
    kfn = pl.kernel(
        _sc_body,
        out_type=[
            jax.ShapeDtypeStruct((NP, D), f32),
            jax.ShapeDtypeStruct((NP, D), f32),
            jax.ShapeDtypeStruct((NP, D), f32),
        ],
        mesh=mesh,
        compiler_params=pltpu.CompilerParams(use_tc_tiling_on_sc=False, needs_layout_passes=False),
        scratch_types=[
            pltpu.VMEM((RAWCH,), jnp.int32),
            pltpu.VMEM((RAWCH,), jnp.int32),
            pltpu.VMEM((CAPE,), jnp.int32),
            pltpu.VMEM((CAPE,), jnp.int32),
            pltpu.VMEM((CAPC, ECH), jnp.int32),
            pltpu.VMEM((ECH, D), f32),
            pltpu.VMEM((ECH, D), f32),
            pltpu.VMEM((RCH, D), f32),
            pltpu.VMEM((RCH, D), f32),
            pltpu.VMEM((TROWS, 16), f32),
            pltpu.VMEM((16, 16), f32),
            pltpu.SMEM((1,), jnp.int32),
            pltpu.SemaphoreType.DMA,
            pltpu.SemaphoreType.DMA,
            pltpu.SemaphoreType.DMA,
            pltpu.SemaphoreType.DMA,
            pltpu.SemaphoreType.REGULAR,
            pltpu.VMEM_SHARED((NLOC + 8, D), f32),
        ],
    )
    hid, _, _ = kfn(x_flat, rawr, rawc, temp_b)
    return hid


def _mlp_body(h_ref, w1_ref, b1_ref, w2_ref, b2_ref, o_ref):
    z = jnp.dot(h_ref[...], w1_ref[...], preferred_element_type=jnp.float32)
    z = jnp.maximum(z + b1_ref[...], 0.0)
    lg = jnp.dot(z, w2_ref[...], preferred_element_type=jnp.float32)
    lg = lg + b2_ref[...]
    m = jnp.max(lg, axis=1, keepdims=True)
    s = jnp.log(jnp.sum(jnp.exp(lg - m), axis=1, keepdims=True))
    o_ref[...] = lg - m - s


def _mlp(hidden, W1, b1, W2, b2):
    BN = 1000
    grid = (N // BN,)
    return pl.pallas_call(
        _mlp_body,
        grid=grid,
        in_specs=[
            pl.BlockSpec((BN, D), lambda i: (i, 0)),
            pl.BlockSpec((D, H), lambda i: (0, 0)),
            pl.BlockSpec((1, H), lambda i: (0, 0)),
            pl.BlockSpec((H, C), lambda i: (0, 0)),
            pl.BlockSpec((1, C), lambda i: (0, 0)),
        ],
        out_specs=pl.BlockSpec((BN, C), lambda i: (i, 0)),
        out_shape=jax.ShapeDtypeStruct((N, C), jnp.float32),
    )(hidden, W1, b1.reshape(1, H), W2, b2.reshape(1, C))


@jax.jit
def kernel(x, edge_index, temp, W1, b1, W2, b2):
    row = edge_index[0]
    col = edge_index[1]
    pad = 16 * 20480 - E
    rawr = jnp.concatenate([row, jnp.zeros((pad,), jnp.int32)])
    rawc = jnp.concatenate([col, jnp.full((pad,), NP, jnp.int32)])
    rawr = rawr.reshape(16, NRAW, RAWCH)
    rawc = rawc.reshape(16, NRAW, RAWCH)
    x_flat = jnp.pad(x, ((0, NP - N), (0, 0)))
    temp_b = jnp.broadcast_to(jnp.pad(temp, (0, 16 - (K + 1)))[:, None],
                              (16, 16)).astype(jnp.float32)
    hid = _propagate(x_flat, rawr, rawc, temp_b)
    return _mlp(hid[:N], W1, b1, W2, b2)


# speedup vs baseline: 1.1030x; 1.1030x over previous
import functools

import jax
import jax.numpy as jnp
from jax import lax
from jax.experimental import pallas as pl
from jax.experimental.pallas import tpu as pltpu
from jax.experimental.pallas import tpu_sc as plsc

N = 10000
E = 320000
D = 128
H = 64
C = 40
K = 10

NP = 10240
NLOC = NP // 2
TROWS = NLOC // 16
RCH = 40
NCH = TROWS // RCH
ECH = 128
CAPC = 104
CAPE = CAPC * ECH
RAWCH = 256
NRAW = 20480 // RAWCH
TRASH = NLOC
ZROW = NP - 1


def _sc_body(x_hbm, rawr_hbm, rawc_hbm, temp_hbm,
             hid_hbm, u_hbm, s_hbm,
             rawbr, rawbc, row_v, colstage, col_v, gbuf, gbuf2,
             abuf, sbuf, dis2b, tempv, offr, gsem, gsem2, xsem,
             acc_sp):
    c = lax.axis_index("c")
    tid = lax.axis_index("s")
    base = tid * TROWS
    gbase = c * NLOC + base
    i32 = jnp.int32

    ones = jnp.full((16,), 1.0, jnp.float32)
    half = jnp.full((16,), 0.5, jnp.float32)

    def babylonian_sqrt(d):
        y = half * (ones + d)
        for _it in range(12):
            y = half * (y + d / y)
        return y

    def cross_sc_sync():
        plsc.subcore_barrier()
        @pl.when(tid == 0)
        def _():
            pltpu.semaphore_signal(xsem, 1, core_index=1 - c)
            pltpu.semaphore_wait(xsem, 1)
        plsc.subcore_barrier()

    pltpu.sync_copy(temp_hbm, tempv)

    zrow_v = jnp.full((16,), ZROW, i32)
    trash_v = jnp.full((16,), TRASH, i32)
    @pl.loop(0, CAPE // 16)
    def _(i):
        sl = pl.ds(i * 16, 16)
        row_v[sl] = zrow_v
        colstage[sl] = trash_v

    lo_v = jnp.full((16,), c * NLOC, i32)
    hi_v = jnp.full((16,), c * NLOC + NLOC, i32)

    offr[0] = jnp.asarray(0, i32)
    @pl.loop(0, NRAW)
    def _bucket_step(q):
        pltpu.sync_copy(rawr_hbm.at[tid, q], rawbr)
        pltpu.sync_copy(rawc_hbm.at[tid, q], rawbc)
        @pl.loop(0, RAWCH // 16)
        def _inner(i):
            sl = pl.ds(i * 16, 16)
            colv = rawbc[sl]
            roww = rawbr[sl]
            m = (colv >= lo_v) & (colv < hi_v)
            cnt = plsc.all_reduce_population_count(m)[0]
            off = offr[0]
            offc = jnp.minimum(off, CAPE - 16)
            plsc.store_compressed(row_v.at[pl.ds(offc, 16)], roww, mask=m)
            plsc.store_compressed(
                colstage.at[pl.ds(offc, 16)], colv - lo_v, mask=m)
            offr[0] = off + cnt

    off = offr[0]
    nch = lax.div(off + (ECH - 1), jnp.asarray(ECH, i32))
    npair = jnp.maximum(lax.div(nch + 1, jnp.asarray(2, i32)), 1)

    @pl.loop(0, CAPC)
    def _(j):
        for g in range(ECH // 16):
            col_v[j, pl.ds(g * 16, 16)] = colstage[pl.ds(j * ECH + g * 16, 16)]

    @pl.loop(0, RCH)
    def _(i):
        z = jnp.zeros((16,), jnp.float32)
        for g in range(D // 16):
            abuf[i, pl.ds(g * 16, 16)] = z
    @pl.loop(0, NCH)
    def _(jj):
        pltpu.sync_copy(abuf, acc_sp.at[pl.ds(base + jj * RCH, RCH)])
    plsc.subcore_barrier()
    @pl.loop(0, ECH)
    def _(i):
        for g in range(D // 16):
            gbuf[i, pl.ds(g * 16, 16)] = ones
    @pl.loop(0, nch)
    def _(j):
        pltpu.sync_copy(gbuf, acc_sp.at[col_v.at[j]], add=True)
    plsc.subcore_barrier()

    t0v = tempv[0, :]
    @pl.loop(0, NCH)
    def _(jj):
        r0l = base + jj * RCH
        r0g = gbase + jj * RCH
        pltpu.sync_copy(acc_sp.at[pl.ds(r0l, RCH)], abuf)
        pltpu.sync_copy(x_hbm.at[pl.ds(r0g, RCH)], sbuf)
        @pl.loop(0, RCH)
        def _(i):
            d = abuf[i, pl.ds(0, 16)] + ones
            d2 = ones / d
            dis2b[jj * RCH + i, :] = d2
            dv = ones / babylonian_sqrt(d)
            for g in range(D // 16):
                sl = pl.ds(g * 16, 16)
                un = dv * sbuf[i, sl]
                abuf[i, sl] = un
                sbuf[i, sl] = t0v * un
        pltpu.sync_copy(abuf, u_hbm.at[pl.ds(r0g, RCH)])
        pltpu.sync_copy(abuf, acc_sp.at[pl.ds(r0l, RCH)])
        pltpu.sync_copy(sbuf, s_hbm.at[pl.ds(r0g, RCH)])
    cross_sc_sync()

    def _gather_start(j, buf, sem):
        pltpu.async_copy(u_hbm.at[row_v.at[pl.ds(j * ECH, ECH)]], buf, sem)

    def _gather_wait(buf, sem):
        pltpu.make_async_copy(
            u_hbm.at[row_v.at[pl.ds(0, ECH)]], buf, sem).wait()

    for k in range(K):
        _gather_start(0, gbuf, gsem)
        _gather_start(1, gbuf2, gsem2)
        @pl.loop(0, npair - 1)
        def _(j2):
            b = 2 * j2
            _gather_wait(gbuf, gsem)
            pltpu.sync_copy(gbuf, acc_sp.at[col_v.at[b]], add=True)
            _gather_start(b + 2, gbuf, gsem)
            _gather_wait(gbuf2, gsem2)
            pltpu.sync_copy(gbuf2, acc_sp.at[col_v.at[b + 1]], add=True)
            _gather_start(b + 3, gbuf2, gsem2)
        bl = 2 * (npair - 1)
        _gather_wait(gbuf, gsem)
        pltpu.sync_copy(gbuf, acc_sp.at[col_v.at[bl]], add=True)
        _gather_wait(gbuf2, gsem2)
        pltpu.sync_copy(gbuf2, acc_sp.at[col_v.at[bl + 1]], add=True)
        plsc.subcore_barrier()

        last = k == K - 1
        tkv = tempv[k + 1, :]
        @pl.loop(0, NCH)
        def _(jj):
            r0l = base + jj * RCH
            r0g = gbase + jj * RCH
            pltpu.sync_copy(acc_sp.at[pl.ds(r0l, RCH)], abuf)
            pltpu.sync_copy(s_hbm.at[pl.ds(r0g, RCH)], sbuf)
            @pl.loop(0, RCH)
            def _(i):
                d2 = dis2b[jj * RCH + i, :]
                if last:
                    iv = babylonian_sqrt(ones / d2)
                for g in range(D // 16):
                    sl = pl.ds(g * 16, 16)
                    un = d2 * abuf[i, sl]
                    s = sbuf[i, sl] + tkv * un
                    if last:
                        s = iv * s
                    else:
                        abuf[i, sl] = un
                    sbuf[i, sl] = s
            if last:
                pltpu.sync_copy(sbuf, hid_hbm.at[pl.ds(r0g, RCH)])
            else:
                pltpu.sync_copy(abuf, u_hbm.at[pl.ds(r0g, RCH)])
                pltpu.sync_copy(abuf, acc_sp.at[pl.ds(r0l, RCH)])
                pltpu.sync_copy(sbuf, s_hbm.at[pl.ds(r0g, RCH)])
        if not last:
            cross_sc_sync()


def _propagate(x_flat, rawr, rawc, temp_b):
    mesh = plsc.VectorSubcoreMesh(core_axis_name="c", subcore_axis_name="s")
    f32 = jnp.float32
    kfn = pl.kernel(
        _sc_body,
        out_type=[
            jax.ShapeDtypeStruct((NP, D), f32),
            jax.ShapeDtypeStruct((NP, D), f32),
            jax.ShapeDtypeStruct((NP, D), f32),
        ],
        mesh=mesh,
        compiler_params=pltpu.CompilerParams(use_tc_tiling_on_sc=False, needs_layout_passes=False),
        scratch_types=[
            pltpu.VMEM((RAWCH,), jnp.int32),
            pltpu.VMEM((RAWCH,), jnp.int32),
            pltpu.VMEM((CAPE,), jnp.int32),
            pltpu.VMEM((CAPE,), jnp.int32),
            pltpu.VMEM((CAPC, ECH), jnp.int32),
            pltpu.VMEM((ECH, D), f32),
            pltpu.VMEM((ECH, D), f32),
            pltpu.VMEM((RCH, D), f32),
            pltpu.VMEM((RCH, D), f32),
            pltpu.VMEM((TROWS, 16), f32),
            pltpu.VMEM((16, 16), f32),
            pltpu.SMEM((1,), jnp.int32),
            pltpu.SemaphoreType.DMA,
            pltpu.SemaphoreType.DMA,
            pltpu.SemaphoreType.REGULAR,
            pltpu.VMEM_SHARED((NLOC + 8, D), f32),
        ],
    )
    hid, _, _ = kfn(x_flat, rawr, rawc, temp_b)
    return hid


def _mlp_body(h_ref, w1_ref, b1_ref, w2_ref, b2_ref, o_ref):
    z = jnp.dot(h_ref[...], w1_ref[...], preferred_element_type=jnp.float32)
    z = jnp.maximum(z + b1_ref[...], 0.0)
    lg = jnp.dot(z, w2_ref[...], preferred_element_type=jnp.float32)
    lg = lg + b2_ref[...]
    m = jnp.max(lg, axis=1, keepdims=True)
    s = jnp.log(jnp.sum(jnp.exp(lg - m), axis=1, keepdims=True))
    o_ref[...] = lg - m - s


def _mlp(hidden, W1, b1, W2, b2):
    BN = 1000
    grid = (N // BN,)
    return pl.pallas_call(
        _mlp_body,
        grid=grid,
        in_specs=[
            pl.BlockSpec((BN, D), lambda i: (i, 0)),
            pl.BlockSpec((D, H), lambda i: (0, 0)),
            pl.BlockSpec((1, H), lambda i: (0, 0)),
            pl.BlockSpec((H, C), lambda i: (0, 0)),
            pl.BlockSpec((1, C), lambda i: (0, 0)),
        ],
        out_specs=pl.BlockSpec((BN, C), lambda i: (i, 0)),
        out_shape=jax.ShapeDtypeStruct((N, C), jnp.float32),
    )(hidden, W1, b1.reshape(1, H), W2, b2.reshape(1, C))


@jax.jit
def kernel(x, edge_index, temp, W1, b1, W2, b2):
    row = edge_index[0]
    col = edge_index[1]
    pad = 16 * 20480 - E
    rawr = jnp.concatenate([row, jnp.zeros((pad,), jnp.int32)])
    rawc = jnp.concatenate([col, jnp.full((pad,), NP, jnp.int32)])
    rawr = rawr.reshape(16, NRAW, RAWCH)
    rawc = rawc.reshape(16, NRAW, RAWCH)
    x_flat = jnp.pad(x, ((0, NP - N), (0, 0)))
    temp_b = jnp.broadcast_to(jnp.pad(temp, (0, 16 - (K + 1)))[:, None],
                              (16, 16)).astype(jnp.float32)
    hid = _propagate(x_flat, rawr, rawc, temp_b)
    return _mlp(hid[:N], W1, b1, W2, b2)
